# baseline (speedup 1.0000x reference)
import jax
import jax.numpy as jnp
from jax import lax
from jax.experimental import pallas as pl
from jax.experimental.pallas import tpu as pltpu
from jax.experimental.pallas import tpu_sc as plsc

N = 10000
E = 320000
NSUB = 16
NCORE = 2
NW = NCORE * NSUB
CH = 64
NCH = 160
NBUF = 4
NHALF = 4
HNCH = NCH // NHALF
NGRP = HNCH // NBUF
EPAD = NW * NCH * CH
NPAD = 10112
STR = NPAD // NSUB
D = 128
F32 = jnp.float32


def _make_sc_agg():
    outs = jax.ShapeDtypeStruct((NCORE, NPAD, D), F32)
    scratch = [
        pltpu.VMEM((HNCH, CH), jnp.int32),
        pltpu.VMEM((HNCH, CH), jnp.int32),
        [pltpu.VMEM((CH, D), F32) for _ in range(NBUF)],
        pltpu.SemaphoreType.DMA,
        pltpu.SemaphoreType.DMA,
        pltpu.VMEM_SHARED((NPAD, D), F32),
    ]
    mesh = plsc.VectorSubcoreMesh(core_axis_name="c", subcore_axis_name="s",
                                  num_cores=NCORE, num_subcores=NSUB)

    def body(tab, srcr, dstr, zrows, out,
             src_v, dst_v, bufs, gsem, ssem, agg_sh):
        c = lax.axis_index("c")
        s = lax.axis_index("s")
        w = c * NSUB + s

        pltpu.sync_copy(zrows, agg_sh.at[pl.ds(s * STR, STR)])
        plsc.subcore_barrier()

        for half in range(NHALF):
            pltpu.sync_copy(srcr.at[w, half], src_v)
            pltpu.sync_copy(dstr.at[w, half], dst_v)
            for i in range(NBUF):
                pltpu.async_copy(tab.at[src_v.at[i]], bufs[i], gsem)

            def grp(g, carry):
                j0 = g * NBUF
                for i in range(NBUF):
                    pltpu.make_async_copy(tab.at[src_v.at[0]],
                                          bufs[i], gsem).wait()
                for i in range(NBUF):
                    pltpu.async_copy(bufs[i], agg_sh.at[dst_v.at[j0 + i]],
                                     ssem, add=True)

                @pl.when(g + 1 < NGRP)
                def _():
                    for i in range(NBUF):
                        pltpu.make_async_copy(bufs[i],
                                              agg_sh.at[dst_v.at[0]],
                                              ssem).wait()
                    for i in range(NBUF):
                        pltpu.async_copy(tab.at[src_v.at[j0 + NBUF + i]],
                                         bufs[i], gsem)

                return carry

            lax.fori_loop(0, NGRP, grp, 0)
            for i in range(NBUF):
                pltpu.make_async_copy(bufs[i], agg_sh.at[dst_v.at[0]],
                                      ssem).wait()
        plsc.subcore_barrier()
        pltpu.sync_copy(agg_sh.at[pl.ds(s * STR, STR)],
                        out.at[c, pl.ds(s * STR, STR)])

    return pl.kernel(body, out_type=outs, mesh=mesh,
                     scratch_types=scratch)


_sc_cache = {}


def _sc_agg():
    if "agg" not in _sc_cache:
        _sc_cache["agg"] = _make_sc_agg()
    return _sc_cache["agg"]


BLK = 1000
GRID = N // BLK


def _mm(a, b):
    return lax.dot_general(a, b, (((1,), (0,)), ((), ())),
                           precision=lax.Precision.HIGHEST,
                           preferred_element_type=F32)


def _row_spec(d):
    return pl.BlockSpec((BLK, d), lambda i: (i, 0))


def _full_spec(r, d):
    return pl.BlockSpec((r, d), lambda i: (0, 0))


def _a1_body(x_ref, wl_ref, wr_ref, xl_ref, xr_ref):
    xl_ref[...] = _mm(x_ref[...], wl_ref[...])
    xr_ref[...] = _mm(x_ref[...], wr_ref[...])


_a1 = pl.pallas_call(
    _a1_body,
    grid=(GRID,),
    in_specs=[_row_spec(128), _full_spec(128, 128), _full_spec(128, 128)],
    out_specs=[_row_spec(128), _row_spec(128)],
    out_shape=[jax.ShapeDtypeStruct((N, 128), F32),
               jax.ShapeDtypeStruct((N, 128), F32)],
)


def _make_mid(emit_xl, dnr):
    def body(*refs):
        if emit_xl:
            (a0, a1, c0, c1, xr, b, g, be, mu, var, wl, wr,
             s_ref, xl_ref, xrn_ref) = refs
        else:
            (a0, a1, c0, c1, xr, b, g, be, mu, var, wr,
             s_ref, xrn_ref) = refs
        agg = a0[...] + a1[...]
        cnt = (c0[...] + c1[...])[:, :1]
        inv = 1.0 / jnp.maximum(cnt, 1.0)
        h = agg * inv + xr[...] + b[...]
        scale = g[...] * lax.rsqrt(var[...] + 1e-5)
        h = (h - mu[...]) * scale + be[...]
        s_out = jnp.maximum(h, 0.0)
        s_ref[...] = s_out
        if emit_xl:
            xl_ref[...] = _mm(s_out, wl[...])
        xrn_ref[...] = _mm(s_out, wr[...])

    in_specs = [_row_spec(128), _row_spec(128), _row_spec(128), _row_spec(128),
                _row_spec(128),
                _full_spec(1, 128), _full_spec(1, 128), _full_spec(1, 128),
                _full_spec(1, 128), _full_spec(1, 128)]
    out_specs = [_row_spec(128)]
    out_shape = [jax.ShapeDtypeStruct((N, 128), F32)]
    if emit_xl:
        in_specs.append(_full_spec(128, 128))
        out_specs.append(_row_spec(128))
        out_shape.append(jax.ShapeDtypeStruct((N, 128), F32))
    in_specs.append(_full_spec(128, dnr))
    out_specs.append(_row_spec(dnr))
    out_shape.append(jax.ShapeDtypeStruct((N, dnr), F32))

    return pl.pallas_call(body, grid=(GRID,), in_specs=in_specs,
                          out_specs=out_specs, out_shape=out_shape)


_mid2 = _make_mid(True, 128)
_mid3 = _make_mid(False, 64)


def _a4_body(a0, a1, c0, c1, xr, b, wl, z_ref, y_ref):
    agg = a0[...] + a1[...]
    cnt = (c0[...] + c1[...])[:, :1]
    inv = 1.0 / jnp.maximum(cnt, 1.0)
    z = _mm(agg * inv, wl[...]) + xr[...] + b[...]
    z_ref[...] = z
    m = jnp.max(z, axis=1, keepdims=True)
    e = jnp.exp(z - m)
    lse = jnp.log(jnp.sum(e, axis=1, keepdims=True)) + m
    y_ref[...] = z - lse


_a4 = pl.pallas_call(
    _a4_body,
    grid=(GRID,),
    in_specs=[_row_spec(128), _row_spec(128), _row_spec(128), _row_spec(128),
              _row_spec(64), _full_spec(1, 64), _full_spec(128, 64)],
    out_specs=[_row_spec(64), _row_spec(64)],
    out_shape=[jax.ShapeDtypeStruct((N, 64), F32),
               jax.ShapeDtypeStruct((N, 64), F32)],
)


def kernel(x, edge_index, W1_l, b1, W1_r, W2_l, b2, W2_r, W3_l, b3, W3_r,
           bn1_gamma, bn1_beta, bn1_mean, bn1_var,
           bn2_gamma, bn2_beta, bn2_mean, bn2_var):
    src = edge_index[0]
    dst = edge_index[1]
    pad = EPAD - E
    srcp = jnp.concatenate([src, jnp.zeros((pad,), jnp.int32)])
    dstp = jnp.concatenate([dst, jnp.full((pad,), N, jnp.int32)])
    srcp = srcp.reshape(NW, NHALF, HNCH, CH)
    dstp = dstp.reshape(NW, NHALF, HNCH, CH)
    zrows = jnp.zeros((STR, D), F32)
    ones_tab = jnp.ones((N, D), F32)

    cc = _sc_agg()(ones_tab, srcp, dstp, zrows)
    c0, c1 = cc[0], cc[1]
    xl1, xr1 = _a1(x, W1_l.T, W1_r.T)
    aa = _sc_agg()(xl1, srcp, dstp, zrows)
    a0, a1 = aa[0], aa[1]
    S1, xl2, xr2 = _mid2(a0, a1, c0, c1, xr1, b1[None],
                         bn1_gamma[None], bn1_beta[None],
                         bn1_mean[None], bn1_var[None],
                         W2_l.T, W2_r.T)
    aa = _sc_agg()(xl2, srcp, dstp, zrows)
    a0, a1 = aa[0], aa[1]
    S2, xr3 = _mid3(a0, a1, c0, c1, xr2, b2[None],
                    bn2_gamma[None], bn2_beta[None],
                    bn2_mean[None], bn2_var[None],
                    W3_r.T)
    aa = _sc_agg()(S2, srcp, dstp, zrows)
    z, y_pred = _a4(aa[0], aa[1], c0, c1, xr3, b3[None], W3_l.T)
    return (z, y_pred, S1, S2)


# speedup vs baseline: 1.0091x; 1.0091x over previous
import jax
import jax.numpy as jnp
from jax import lax
from jax.experimental import pallas as pl
from jax.experimental.pallas import tpu as pltpu
from jax.experimental.pallas import tpu_sc as plsc

N = 10000
E = 320000
NSUB = 16
NCORE = 2
NW = NCORE * NSUB
CH = 128
NCH = 80
NBUF = 2
NHALF = 2
HNCH = NCH // NHALF
NGRP = HNCH // NBUF
EPAD = NW * NCH * CH
NPAD = 10112
STR = NPAD // NSUB
D = 128
F32 = jnp.float32


def _make_sc_agg():
    outs = jax.ShapeDtypeStruct((NCORE, NPAD, D), F32)
    scratch = [
        pltpu.VMEM((HNCH, CH), jnp.int32),
        pltpu.VMEM((HNCH, CH), jnp.int32),
        [pltpu.VMEM((CH, D), F32) for _ in range(NBUF)],
        pltpu.SemaphoreType.DMA,
        pltpu.SemaphoreType.DMA,
        pltpu.VMEM_SHARED((NPAD, D), F32),
    ]
    mesh = plsc.VectorSubcoreMesh(core_axis_name="c", subcore_axis_name="s",
                                  num_cores=NCORE, num_subcores=NSUB)

    def body(tab, srcr, dstr, zrows, out,
             src_v, dst_v, bufs, gsem, ssem, agg_sh):
        c = lax.axis_index("c")
        s = lax.axis_index("s")
        w = c * NSUB + s

        pltpu.sync_copy(zrows, agg_sh.at[pl.ds(s * STR, STR)])
        plsc.subcore_barrier()

        for half in range(NHALF):
            pltpu.sync_copy(srcr.at[w, half], src_v)
            pltpu.sync_copy(dstr.at[w, half], dst_v)
            for i in range(NBUF):
                pltpu.async_copy(tab.at[src_v.at[i]], bufs[i], gsem)

            def grp(g, carry):
                j0 = g * NBUF
                for i in range(NBUF):
                    pltpu.make_async_copy(tab.at[src_v.at[0]],
                                          bufs[i], gsem).wait()
                for i in range(NBUF):
                    pltpu.async_copy(bufs[i], agg_sh.at[dst_v.at[j0 + i]],
                                     ssem, add=True)

                @pl.when(g + 1 < NGRP)
                def _():
                    for i in range(NBUF):
                        pltpu.make_async_copy(bufs[i],
                                              agg_sh.at[dst_v.at[0]],
                                              ssem).wait()
                    for i in range(NBUF):
                        pltpu.async_copy(tab.at[src_v.at[j0 + NBUF + i]],
                                         bufs[i], gsem)

                return carry

            lax.fori_loop(0, NGRP, grp, 0)
            for i in range(NBUF):
                pltpu.make_async_copy(bufs[i], agg_sh.at[dst_v.at[0]],
                                      ssem).wait()
        plsc.subcore_barrier()
        pltpu.sync_copy(agg_sh.at[pl.ds(s * STR, STR)],
                        out.at[c, pl.ds(s * STR, STR)])

    return pl.kernel(body, out_type=outs, mesh=mesh,
                     scratch_types=scratch)


_sc_cache = {}


def _sc_agg():
    if "agg" not in _sc_cache:
        _sc_cache["agg"] = _make_sc_agg()
    return _sc_cache["agg"]


BLK = 1000
GRID = N // BLK


def _mm(a, b):
    return lax.dot_general(a, b, (((1,), (0,)), ((), ())),
                           precision=lax.Precision.HIGHEST,
                           preferred_element_type=F32)


def _row_spec(d):
    return pl.BlockSpec((BLK, d), lambda i: (i, 0))


def _full_spec(r, d):
    return pl.BlockSpec((r, d), lambda i: (0, 0))


def _a1_body(x_ref, wl_ref, wr_ref, xl_ref, xr_ref):
    xl_ref[...] = _mm(x_ref[...], wl_ref[...])
    xr_ref[...] = _mm(x_ref[...], wr_ref[...])


_a1 = pl.pallas_call(
    _a1_body,
    grid=(GRID,),
    in_specs=[_row_spec(128), _full_spec(128, 128), _full_spec(128, 128)],
    out_specs=[_row_spec(128), _row_spec(128)],
    out_shape=[jax.ShapeDtypeStruct((N, 128), F32),
               jax.ShapeDtypeStruct((N, 128), F32)],
)


def _make_mid(emit_xl, dnr):
    def body(*refs):
        if emit_xl:
            (a0, a1, c0, c1, xr, b, g, be, mu, var, wl, wr,
             s_ref, xl_ref, xrn_ref) = refs
        else:
            (a0, a1, c0, c1, xr, b, g, be, mu, var, wr,
             s_ref, xrn_ref) = refs
        agg = a0[...] + a1[...]
        cnt = (c0[...] + c1[...])[:, :1]
        inv = 1.0 / jnp.maximum(cnt, 1.0)
        h = agg * inv + xr[...] + b[...]
        scale = g[...] * lax.rsqrt(var[...] + 1e-5)
        h = (h - mu[...]) * scale + be[...]
        s_out = jnp.maximum(h, 0.0)
        s_ref[...] = s_out
        if emit_xl:
            xl_ref[...] = _mm(s_out, wl[...])
        xrn_ref[...] = _mm(s_out, wr[...])

    in_specs = [_row_spec(128), _row_spec(128), _row_spec(128), _row_spec(128),
                _row_spec(128),
                _full_spec(1, 128), _full_spec(1, 128), _full_spec(1, 128),
                _full_spec(1, 128), _full_spec(1, 128)]
    out_specs = [_row_spec(128)]
    out_shape = [jax.ShapeDtypeStruct((N, 128), F32)]
    if emit_xl:
        in_specs.append(_full_spec(128, 128))
        out_specs.append(_row_spec(128))
        out_shape.append(jax.ShapeDtypeStruct((N, 128), F32))
    in_specs.append(_full_spec(128, dnr))
    out_specs.append(_row_spec(dnr))
    out_shape.append(jax.ShapeDtypeStruct((N, dnr), F32))

    return pl.pallas_call(body, grid=(GRID,), in_specs=in_specs,
                          out_specs=out_specs, out_shape=out_shape)


_mid2 = _make_mid(True, 128)
_mid3 = _make_mid(False, 64)


def _a4_body(a0, a1, c0, c1, xr, b, wl, z_ref, y_ref):
    agg = a0[...] + a1[...]
    cnt = (c0[...] + c1[...])[:, :1]
    inv = 1.0 / jnp.maximum(cnt, 1.0)
    z = _mm(agg * inv, wl[...]) + xr[...] + b[...]
    z_ref[...] = z
    m = jnp.max(z, axis=1, keepdims=True)
    e = jnp.exp(z - m)
    lse = jnp.log(jnp.sum(e, axis=1, keepdims=True)) + m
    y_ref[...] = z - lse


_a4 = pl.pallas_call(
    _a4_body,
    grid=(GRID,),
    in_specs=[_row_spec(128), _row_spec(128), _row_spec(128), _row_spec(128),
              _row_spec(64), _full_spec(1, 64), _full_spec(128, 64)],
    out_specs=[_row_spec(64), _row_spec(64)],
    out_shape=[jax.ShapeDtypeStruct((N, 64), F32),
               jax.ShapeDtypeStruct((N, 64), F32)],
)


def kernel(x, edge_index, W1_l, b1, W1_r, W2_l, b2, W2_r, W3_l, b3, W3_r,
           bn1_gamma, bn1_beta, bn1_mean, bn1_var,
           bn2_gamma, bn2_beta, bn2_mean, bn2_var):
    src = edge_index[0]
    dst = edge_index[1]
    pad = EPAD - E
    srcp = jnp.concatenate([src, jnp.zeros((pad,), jnp.int32)])
    dstp = jnp.concatenate([dst, jnp.full((pad,), N, jnp.int32)])
    srcp = srcp.reshape(NW, NHALF, HNCH, CH)
    dstp = dstp.reshape(NW, NHALF, HNCH, CH)
    zrows = jnp.zeros((STR, D), F32)
    ones_tab = jnp.ones((N, D), F32)

    cc = _sc_agg()(ones_tab, srcp, dstp, zrows)
    c0, c1 = cc[0], cc[1]
    xl1, xr1 = _a1(x, W1_l.T, W1_r.T)
    aa = _sc_agg()(xl1, srcp, dstp, zrows)
    a0, a1 = aa[0], aa[1]
    S1, xl2, xr2 = _mid2(a0, a1, c0, c1, xr1, b1[None],
                         bn1_gamma[None], bn1_beta[None],
                         bn1_mean[None], bn1_var[None],
                         W2_l.T, W2_r.T)
    aa = _sc_agg()(xl2, srcp, dstp, zrows)
    a0, a1 = aa[0], aa[1]
    S2, xr3 = _mid3(a0, a1, c0, c1, xr2, b2[None],
                    bn2_gamma[None], bn2_beta[None],
                    bn2_mean[None], bn2_var[None],
                    W3_r.T)
    aa = _sc_agg()(S2, srcp, dstp, zrows)
    z, y_pred = _a4(aa[0], aa[1], c0, c1, xr3, b3[None], W3_l.T)
    return (z, y_pred, S1, S2)


# speedup vs baseline: 3.8984x; 3.8633x over previous
import jax
import jax.numpy as jnp
from jax import lax
from jax.experimental import pallas as pl
from jax.experimental.pallas import tpu as pltpu
from jax.experimental.pallas import tpu_sc as plsc

N = 10000
E = 320000
NSUB = 16
NCORE = 2
NW = NCORE * NSUB
CH = 128
NCH = 80
NHALF = 2
HNCH = NCH // NHALF
HPAIR = HNCH // 2
EPAD = NW * NCH * CH
NPAD = 10112
STR = NPAD // NSUB
D = 128
F32 = jnp.float32


def _make_sc_agg():
    outs = jax.ShapeDtypeStruct((NCORE, NPAD, D), F32)
    scratch = [
        pltpu.VMEM((HNCH, CH), jnp.int32),
        pltpu.VMEM((HNCH, CH), jnp.int32),
        pltpu.VMEM((CH, D), F32),
        pltpu.VMEM((CH, D), F32),
        pltpu.SemaphoreType.DMA,
        pltpu.SemaphoreType.DMA,
        pltpu.VMEM_SHARED((NPAD, D), F32),
    ]
    mesh = plsc.VectorSubcoreMesh(core_axis_name="c", subcore_axis_name="s",
                                  num_cores=NCORE, num_subcores=NSUB)

    def body(tab, srcr, dstr, zrows, out,
             src_v, dst_v, r_a, r_b, sem_a, sem_b, agg_sh):
        c = lax.axis_index("c")
        s = lax.axis_index("s")
        w = c * NSUB + s

        pltpu.sync_copy(zrows, agg_sh.at[pl.ds(s * STR, STR)])
        plsc.subcore_barrier()

        for half in range(NHALF):
            pltpu.sync_copy(srcr.at[w, half], src_v)
            pltpu.sync_copy(dstr.at[w, half], dst_v)
            pltpu.async_copy(tab.at[src_v.at[0]], r_a, sem_a)
            pltpu.async_copy(tab.at[src_v.at[1]], r_b, sem_b)

            def step(p, carry):
                j = 2 * p
                pltpu.make_async_copy(tab.at[src_v.at[0]], r_a, sem_a).wait()
                pltpu.sync_copy(r_a, agg_sh.at[dst_v.at[j]], add=True)

                @pl.when(p + 1 < HPAIR)
                def _():
                    pltpu.async_copy(tab.at[src_v.at[j + 2]], r_a, sem_a)

                pltpu.make_async_copy(tab.at[src_v.at[1]], r_b, sem_b).wait()
                pltpu.sync_copy(r_b, agg_sh.at[dst_v.at[j + 1]], add=True)

                @pl.when(p + 1 < HPAIR)
                def _():
                    pltpu.async_copy(tab.at[src_v.at[j + 3]], r_b, sem_b)

                return carry

            lax.fori_loop(0, HPAIR, step, 0)
        plsc.subcore_barrier()
        pltpu.sync_copy(agg_sh.at[pl.ds(s * STR, STR)],
                        out.at[c, pl.ds(s * STR, STR)])

    return pl.kernel(body, out_type=outs, mesh=mesh,
                     scratch_types=scratch)


_sc_cache = {}


def _sc_agg():
    if "agg" not in _sc_cache:
        _sc_cache["agg"] = _make_sc_agg()
    return _sc_cache["agg"]


BLK = 1000
GRID = N // BLK


def _mm(a, b):
    return lax.dot_general(a, b, (((1,), (0,)), ((), ())),
                           precision=lax.Precision.HIGHEST,
                           preferred_element_type=F32)


def _row_spec(d):
    return pl.BlockSpec((BLK, d), lambda i: (i, 0))


def _full_spec(r, d):
    return pl.BlockSpec((r, d), lambda i: (0, 0))


def _a1_body(x_ref, wl_ref, wr_ref, xl_ref, xr_ref):
    xl_ref[...] = _mm(x_ref[...], wl_ref[...])
    xr_ref[...] = _mm(x_ref[...], wr_ref[...])


_a1 = pl.pallas_call(
    _a1_body,
    grid=(GRID,),
    in_specs=[_row_spec(128), _full_spec(128, 128), _full_spec(128, 128)],
    out_specs=[_row_spec(128), _row_spec(128)],
    out_shape=[jax.ShapeDtypeStruct((N, 128), F32),
               jax.ShapeDtypeStruct((N, 128), F32)],
)


def _make_mid(emit_xl, dnr):
    def body(*refs):
        if emit_xl:
            (a0, a1, c0, c1, xr, b, g, be, mu, var, wl, wr,
             s_ref, xl_ref, xrn_ref) = refs
        else:
            (a0, a1, c0, c1, xr, b, g, be, mu, var, wr,
             s_ref, xrn_ref) = refs
        agg = a0[...] + a1[...]
        cnt = (c0[...] + c1[...])[:, :1]
        inv = 1.0 / jnp.maximum(cnt, 1.0)
        h = agg * inv + xr[...] + b[...]
        scale = g[...] * lax.rsqrt(var[...] + 1e-5)
        h = (h - mu[...]) * scale + be[...]
        s_out = jnp.maximum(h, 0.0)
        s_ref[...] = s_out
        if emit_xl:
            xl_ref[...] = _mm(s_out, wl[...])
        xrn_ref[...] = _mm(s_out, wr[...])

    in_specs = [_row_spec(128), _row_spec(128), _row_spec(128), _row_spec(128),
                _row_spec(128),
                _full_spec(1, 128), _full_spec(1, 128), _full_spec(1, 128),
                _full_spec(1, 128), _full_spec(1, 128)]
    out_specs = [_row_spec(128)]
    out_shape = [jax.ShapeDtypeStruct((N, 128), F32)]
    if emit_xl:
        in_specs.append(_full_spec(128, 128))
        out_specs.append(_row_spec(128))
        out_shape.append(jax.ShapeDtypeStruct((N, 128), F32))
    in_specs.append(_full_spec(128, dnr))
    out_specs.append(_row_spec(dnr))
    out_shape.append(jax.ShapeDtypeStruct((N, dnr), F32))

    return pl.pallas_call(body, grid=(GRID,), in_specs=in_specs,
                          out_specs=out_specs, out_shape=out_shape)


_mid2 = _make_mid(True, 128)
_mid3 = _make_mid(False, 64)


def _a4_body(a0, a1, c0, c1, xr, b, wl, z_ref, y_ref):
    agg = a0[...] + a1[...]
    cnt = (c0[...] + c1[...])[:, :1]
    inv = 1.0 / jnp.maximum(cnt, 1.0)
    z = _mm(agg * inv, wl[...]) + xr[...] + b[...]
    z_ref[...] = z
    m = jnp.max(z, axis=1, keepdims=True)
    e = jnp.exp(z - m)
    lse = jnp.log(jnp.sum(e, axis=1, keepdims=True)) + m
    y_ref[...] = z - lse


_a4 = pl.pallas_call(
    _a4_body,
    grid=(GRID,),
    in_specs=[_row_spec(128), _row_spec(128), _row_spec(128), _row_spec(128),
              _row_spec(64), _full_spec(1, 64), _full_spec(128, 64)],
    out_specs=[_row_spec(64), _row_spec(64)],
    out_shape=[jax.ShapeDtypeStruct((N, 64), F32),
               jax.ShapeDtypeStruct((N, 64), F32)],
)


def kernel(x, edge_index, W1_l, b1, W1_r, W2_l, b2, W2_r, W3_l, b3, W3_r,
           bn1_gamma, bn1_beta, bn1_mean, bn1_var,
           bn2_gamma, bn2_beta, bn2_mean, bn2_var):
    src = edge_index[0]
    dst = edge_index[1]
    pad = EPAD - E
    pad_src = (jnp.arange(pad, dtype=jnp.int32) * 37) % N
    pad_dst = N + (jnp.arange(pad, dtype=jnp.int32) % (NPAD - N))
    srcp = jnp.concatenate([src, pad_src])
    dstp = jnp.concatenate([dst, pad_dst])
    srcp = srcp.reshape(NW, NHALF, HNCH, CH)
    dstp = dstp.reshape(NW, NHALF, HNCH, CH)
    zrows = jnp.zeros((STR, D), F32)
    ones_tab = jnp.ones((N, D), F32)

    cc = _sc_agg()(ones_tab, srcp, dstp, zrows)
    c0, c1 = cc[0], cc[1]
    xl1, xr1 = _a1(x, W1_l.T, W1_r.T)
    aa = _sc_agg()(xl1, srcp, dstp, zrows)
    a0, a1 = aa[0], aa[1]
    S1, xl2, xr2 = _mid2(a0, a1, c0, c1, xr1, b1[None],
                         bn1_gamma[None], bn1_beta[None],
                         bn1_mean[None], bn1_var[None],
                         W2_l.T, W2_r.T)
    aa = _sc_agg()(xl2, srcp, dstp, zrows)
    a0, a1 = aa[0], aa[1]
    S2, xr3 = _mid3(a0, a1, c0, c1, xr2, b2[None],
                    bn2_gamma[None], bn2_beta[None],
                    bn2_mean[None], bn2_var[None],
                    W3_r.T)
    aa = _sc_agg()(S2, srcp, dstp, zrows)
    z, y_pred = _a4(aa[0], aa[1], c0, c1, xr3, b3[None], W3_l.T)
    return (z, y_pred, S1, S2)


# speedup vs baseline: 4.0402x; 1.0364x over previous
import jax
import jax.numpy as jnp
from jax import lax
from jax.experimental import pallas as pl
from jax.experimental.pallas import tpu as pltpu
from jax.experimental.pallas import tpu_sc as plsc

N = 10000
E = 320000
NSUB = 16
NCORE = 2
NW = NCORE * NSUB
CH = 128
NCH = 80
NHALF = 2
HNCH = NCH // NHALF
HPAIR = HNCH // 2
EPAD = NW * NCH * CH
NPAD = 10112
STR = NPAD // NSUB
D = 128
F32 = jnp.float32


def _make_sc_agg(with_cnt):
    if with_cnt:
        outs = (jax.ShapeDtypeStruct((NCORE, NPAD, D), F32),
                jax.ShapeDtypeStruct((NCORE, NPAD, D), F32))
    else:
        outs = jax.ShapeDtypeStruct((NCORE, NPAD, D), F32)
    scratch = [
        pltpu.VMEM((HNCH, CH), jnp.int32),
        pltpu.VMEM((HNCH, CH), jnp.int32),
        pltpu.VMEM((CH, D), F32),
        pltpu.VMEM((CH, D), F32),
        pltpu.SemaphoreType.DMA,
        pltpu.SemaphoreType.DMA,
        pltpu.VMEM_SHARED((NPAD, D), F32),
    ]
    mesh = plsc.VectorSubcoreMesh(core_axis_name="c", subcore_axis_name="s",
                                  num_cores=NCORE, num_subcores=NSUB)

    def body(*refs):
        if with_cnt:
            (tab, srcr, dstr, zrows, ones_hbm, out, cnt_out,
             src_v, dst_v, r_a, r_b, sem_a, sem_b, agg_sh) = refs
        else:
            (tab, srcr, dstr, zrows, out,
             src_v, dst_v, r_a, r_b, sem_a, sem_b, agg_sh) = refs
        c = lax.axis_index("c")
        s = lax.axis_index("s")
        w = c * NSUB + s

        pltpu.sync_copy(zrows, agg_sh.at[pl.ds(s * STR, STR)])

        if with_cnt:
            pltpu.sync_copy(ones_hbm, r_a)
            plsc.subcore_barrier()
            for half in range(NHALF):
                pltpu.sync_copy(dstr.at[w, half], dst_v)

                def cstep(j, carry):
                    pltpu.sync_copy(r_a, agg_sh.at[dst_v.at[j]], add=True)
                    return carry

                lax.fori_loop(0, HNCH, cstep, 0)
            plsc.subcore_barrier()
            pltpu.sync_copy(agg_sh.at[pl.ds(s * STR, STR)],
                            cnt_out.at[c, pl.ds(s * STR, STR)])
            pltpu.sync_copy(zrows, agg_sh.at[pl.ds(s * STR, STR)])
        plsc.subcore_barrier()

        for half in range(NHALF):
            pltpu.sync_copy(srcr.at[w, half], src_v)
            pltpu.sync_copy(dstr.at[w, half], dst_v)
            pltpu.async_copy(tab.at[src_v.at[0]], r_a, sem_a)
            pltpu.async_copy(tab.at[src_v.at[1]], r_b, sem_b)

            def step(p, carry):
                j = 2 * p
                pltpu.make_async_copy(tab.at[src_v.at[0]], r_a, sem_a).wait()
                pltpu.sync_copy(r_a, agg_sh.at[dst_v.at[j]], add=True)

                @pl.when(p + 1 < HPAIR)
                def _():
                    pltpu.async_copy(tab.at[src_v.at[j + 2]], r_a, sem_a)

                pltpu.make_async_copy(tab.at[src_v.at[1]], r_b, sem_b).wait()
                pltpu.sync_copy(r_b, agg_sh.at[dst_v.at[j + 1]], add=True)

                @pl.when(p + 1 < HPAIR)
                def _():
                    pltpu.async_copy(tab.at[src_v.at[j + 3]], r_b, sem_b)

                return carry

            lax.fori_loop(0, HPAIR, step, 0)
        plsc.subcore_barrier()
        pltpu.sync_copy(agg_sh.at[pl.ds(s * STR, STR)],
                        out.at[c, pl.ds(s * STR, STR)])

    return pl.kernel(body, out_type=outs, mesh=mesh,
                     scratch_types=scratch)


_sc_cache = {}


def _sc_agg(with_cnt=False):
    if with_cnt not in _sc_cache:
        _sc_cache[with_cnt] = _make_sc_agg(with_cnt)
    return _sc_cache[with_cnt]


BLK = 1000
GRID = N // BLK


def _mm(a, b):
    return lax.dot_general(a, b, (((1,), (0,)), ((), ())),
                           precision=lax.Precision.HIGHEST,
                           preferred_element_type=F32)


def _row_spec(d):
    return pl.BlockSpec((BLK, d), lambda i: (i, 0))


def _full_spec(r, d):
    return pl.BlockSpec((r, d), lambda i: (0, 0))


def _a1_body(x_ref, wl_ref, wr_ref, xl_ref, xr_ref):
    xl_ref[...] = _mm(x_ref[...], wl_ref[...])
    xr_ref[...] = _mm(x_ref[...], wr_ref[...])


_a1 = pl.pallas_call(
    _a1_body,
    grid=(GRID,),
    in_specs=[_row_spec(128), _full_spec(128, 128), _full_spec(128, 128)],
    out_specs=[_row_spec(128), _row_spec(128)],
    out_shape=[jax.ShapeDtypeStruct((N, 128), F32),
               jax.ShapeDtypeStruct((N, 128), F32)],
)


def _make_mid(emit_xl, dnr):
    def body(*refs):
        if emit_xl:
            (a0, a1, c0, c1, xr, b, g, be, mu, var, wl, wr,
             s_ref, xl_ref, xrn_ref) = refs
        else:
            (a0, a1, c0, c1, xr, b, g, be, mu, var, wr,
             s_ref, xrn_ref) = refs
        agg = a0[...] + a1[...]
        cnt = (c0[...] + c1[...])[:, :1]
        inv = 1.0 / jnp.maximum(cnt, 1.0)
        h = agg * inv + xr[...] + b[...]
        scale = g[...] * lax.rsqrt(var[...] + 1e-5)
        h = (h - mu[...]) * scale + be[...]
        s_out = jnp.maximum(h, 0.0)
        s_ref[...] = s_out
        if emit_xl:
            xl_ref[...] = _mm(s_out, wl[...])
        xrn_ref[...] = _mm(s_out, wr[...])

    in_specs = [_row_spec(128), _row_spec(128), _row_spec(128), _row_spec(128),
                _row_spec(128),
                _full_spec(1, 128), _full_spec(1, 128), _full_spec(1, 128),
                _full_spec(1, 128), _full_spec(1, 128)]
    out_specs = [_row_spec(128)]
    out_shape = [jax.ShapeDtypeStruct((N, 128), F32)]
    if emit_xl:
        in_specs.append(_full_spec(128, 128))
        out_specs.append(_row_spec(128))
        out_shape.append(jax.ShapeDtypeStruct((N, 128), F32))
    in_specs.append(_full_spec(128, dnr))
    out_specs.append(_row_spec(dnr))
    out_shape.append(jax.ShapeDtypeStruct((N, dnr), F32))

    return pl.pallas_call(body, grid=(GRID,), in_specs=in_specs,
                          out_specs=out_specs, out_shape=out_shape)


_mid2 = _make_mid(True, 128)
_mid3 = _make_mid(False, 64)


def _a4_body(a0, a1, c0, c1, xr, b, wl, z_ref, y_ref):
    agg = a0[...] + a1[...]
    cnt = (c0[...] + c1[...])[:, :1]
    inv = 1.0 / jnp.maximum(cnt, 1.0)
    z = _mm(agg * inv, wl[...]) + xr[...] + b[...]
    z_ref[...] = z
    m = jnp.max(z, axis=1, keepdims=True)
    e = jnp.exp(z - m)
    lse = jnp.log(jnp.sum(e, axis=1, keepdims=True)) + m
    y_ref[...] = z - lse


_a4 = pl.pallas_call(
    _a4_body,
    grid=(GRID,),
    in_specs=[_row_spec(128), _row_spec(128), _row_spec(128), _row_spec(128),
              _row_spec(64), _full_spec(1, 64), _full_spec(128, 64)],
    out_specs=[_row_spec(64), _row_spec(64)],
    out_shape=[jax.ShapeDtypeStruct((N, 64), F32),
               jax.ShapeDtypeStruct((N, 64), F32)],
)


def kernel(x, edge_index, W1_l, b1, W1_r, W2_l, b2, W2_r, W3_l, b3, W3_r,
           bn1_gamma, bn1_beta, bn1_mean, bn1_var,
           bn2_gamma, bn2_beta, bn2_mean, bn2_var):
    src = edge_index[0]
    dst = edge_index[1]
    pad = EPAD - E
    pad_src = (jnp.arange(pad, dtype=jnp.int32) * 37) % N
    pad_dst = N + (jnp.arange(pad, dtype=jnp.int32) % (NPAD - N))
    srcp = jnp.concatenate([src, pad_src])
    dstp = jnp.concatenate([dst, pad_dst])
    srcp = srcp.reshape(NW, NHALF, HNCH, CH)
    dstp = dstp.reshape(NW, NHALF, HNCH, CH)
    zrows = jnp.zeros((STR, D), F32)
    ones128 = jnp.ones((CH, D), F32)

    xl1, xr1 = _a1(x, W1_l.T, W1_r.T)
    aa, cc = _sc_agg(True)(xl1, srcp, dstp, zrows, ones128)
    c0, c1 = cc[0], cc[1]
    a0, a1 = aa[0], aa[1]
    S1, xl2, xr2 = _mid2(a0, a1, c0, c1, xr1, b1[None],
                         bn1_gamma[None], bn1_beta[None],
                         bn1_mean[None], bn1_var[None],
                         W2_l.T, W2_r.T)
    aa = _sc_agg()(xl2, srcp, dstp, zrows)
    a0, a1 = aa[0], aa[1]
    S2, xr3 = _mid3(a0, a1, c0, c1, xr2, b2[None],
                    bn2_gamma[None], bn2_beta[None],
                    bn2_mean[None], bn2_var[None],
                    W3_r.T)
    aa = _sc_agg()(S2, srcp, dstp, zrows)
    z, y_pred = _a4(aa[0], aa[1], c0, c1, xr3, b3[None], W3_l.T)
    return (z, y_pred, S1, S2)


# speedup vs baseline: 4.1998x; 1.0395x over previous
import jax
import jax.numpy as jnp
from jax import lax
from jax.experimental import pallas as pl
from jax.experimental.pallas import tpu as pltpu
from jax.experimental.pallas import tpu_sc as plsc

N = 10000
E = 320000
NSUB = 16
NCORE = 2
NW = NCORE * NSUB
CH = 128
NCH = 80
NHALF = 2
HNCH = NCH // NHALF
HPAIR = HNCH // 2
EPAD = NW * NCH * CH
NPAD = 10112
STR = NPAD // NSUB
D = 128
F32 = jnp.float32


def _make_sc_agg(with_cnt):
    if with_cnt:
        outs = (jax.ShapeDtypeStruct((NCORE, NPAD, D), F32),
                jax.ShapeDtypeStruct((NCORE, NPAD, D), F32))
    else:
        outs = jax.ShapeDtypeStruct((NCORE, NPAD, D), F32)
    scratch = [
        pltpu.VMEM((HNCH, CH), jnp.int32),
        pltpu.VMEM((HNCH, CH), jnp.int32),
        pltpu.VMEM((CH, D), F32),
        pltpu.VMEM((CH, D), F32),
        pltpu.SemaphoreType.DMA,
        pltpu.SemaphoreType.DMA,
        pltpu.VMEM_SHARED((NPAD, D), F32),
    ]
    mesh = plsc.VectorSubcoreMesh(core_axis_name="c", subcore_axis_name="s",
                                  num_cores=NCORE, num_subcores=NSUB)

    def body(*refs):
        if with_cnt:
            (tab, srcr, dstr, zrows, ones_hbm, out, cnt_out,
             src_v, dst_v, r_a, r_b, sem_a, sem_b, agg_sh) = refs
        else:
            (tab, srcr, dstr, zrows, out,
             src_v, dst_v, r_a, r_b, sem_a, sem_b, agg_sh) = refs
        c = lax.axis_index("c")
        s = lax.axis_index("s")
        w = c * NSUB + s

        pltpu.sync_copy(zrows, agg_sh.at[pl.ds(s * STR, STR)])

        if with_cnt:
            pltpu.sync_copy(ones_hbm, r_a)
            plsc.subcore_barrier()
            for half in range(NHALF):
                pltpu.sync_copy(dstr.at[w, half], dst_v)

                def cstep(j, carry):
                    pltpu.sync_copy(r_a, agg_sh.at[dst_v.at[j]], add=True)
                    return carry

                lax.fori_loop(0, HNCH, cstep, 0)
            plsc.subcore_barrier()
            pltpu.sync_copy(agg_sh.at[pl.ds(s * STR, STR)],
                            cnt_out.at[c, pl.ds(s * STR, STR)])
            pltpu.sync_copy(zrows, agg_sh.at[pl.ds(s * STR, STR)])
        plsc.subcore_barrier()

        for half in range(NHALF):
            pltpu.sync_copy(srcr.at[w, half], src_v)
            pltpu.sync_copy(dstr.at[w, half], dst_v)
            pltpu.async_copy(tab.at[src_v.at[0]], r_a, sem_a)
            pltpu.async_copy(tab.at[src_v.at[1]], r_b, sem_b)

            def step(p, carry):
                j = 2 * p
                pltpu.make_async_copy(tab.at[src_v.at[0]], r_a, sem_a).wait()
                pltpu.sync_copy(r_a, agg_sh.at[dst_v.at[j]], add=True)

                @pl.when(p + 1 < HPAIR)
                def _():
                    pltpu.async_copy(tab.at[src_v.at[j + 2]], r_a, sem_a)

                pltpu.make_async_copy(tab.at[src_v.at[1]], r_b, sem_b).wait()
                pltpu.sync_copy(r_b, agg_sh.at[dst_v.at[j + 1]], add=True)

                @pl.when(p + 1 < HPAIR)
                def _():
                    pltpu.async_copy(tab.at[src_v.at[j + 3]], r_b, sem_b)

                return carry

            lax.fori_loop(0, HPAIR, step, 0)
        plsc.subcore_barrier()
        pltpu.sync_copy(agg_sh.at[pl.ds(s * STR, STR)],
                        out.at[c, pl.ds(s * STR, STR)])

    return pl.kernel(body, out_type=outs, mesh=mesh,
                     scratch_types=scratch)


_sc_cache = {}


def _sc_agg(with_cnt=False):
    if with_cnt not in _sc_cache:
        _sc_cache[with_cnt] = _make_sc_agg(with_cnt)
    return _sc_cache[with_cnt]


BLK = 2000
GRID = N // BLK


def _mm(a, b):
    return lax.dot_general(a, b, (((1,), (0,)), ((), ())),
                           precision=lax.Precision.HIGHEST,
                           preferred_element_type=F32)


def _row_spec(d):
    return pl.BlockSpec((BLK, d), lambda i: (i, 0))


def _full_spec(r, d):
    return pl.BlockSpec((r, d), lambda i: (0, 0))


def _a1_body(x_ref, wl_ref, wr_ref, xl_ref, xr_ref):
    xl_ref[...] = _mm(x_ref[...], wl_ref[...])
    xr_ref[...] = _mm(x_ref[...], wr_ref[...])


_a1 = pl.pallas_call(
    _a1_body,
    grid=(GRID,),
    in_specs=[_row_spec(128), _full_spec(128, 128), _full_spec(128, 128)],
    out_specs=[_row_spec(128), _row_spec(128)],
    out_shape=[jax.ShapeDtypeStruct((N, 128), F32),
               jax.ShapeDtypeStruct((N, 128), F32)],
)


def _make_mid(emit_xl, dnr):
    def body(*refs):
        if emit_xl:
            (a0, a1, c0, c1, xr, b, g, be, mu, var, wl, wr,
             s_ref, xl_ref, xrn_ref) = refs
        else:
            (a0, a1, c0, c1, xr, b, g, be, mu, var, wr,
             s_ref, xrn_ref) = refs
        agg = a0[...] + a1[...]
        cnt = (c0[...] + c1[...])[:, :1]
        inv = 1.0 / jnp.maximum(cnt, 1.0)
        h = agg * inv + xr[...] + b[...]
        scale = g[...] * lax.rsqrt(var[...] + 1e-5)
        h = (h - mu[...]) * scale + be[...]
        s_out = jnp.maximum(h, 0.0)
        s_ref[...] = s_out
        if emit_xl:
            xl_ref[...] = _mm(s_out, wl[...])
        xrn_ref[...] = _mm(s_out, wr[...])

    in_specs = [_row_spec(128), _row_spec(128), _row_spec(8), _row_spec(8),
                _row_spec(128),
                _full_spec(1, 128), _full_spec(1, 128), _full_spec(1, 128),
                _full_spec(1, 128), _full_spec(1, 128)]
    out_specs = [_row_spec(128)]
    out_shape = [jax.ShapeDtypeStruct((N, 128), F32)]
    if emit_xl:
        in_specs.append(_full_spec(128, 128))
        out_specs.append(_row_spec(128))
        out_shape.append(jax.ShapeDtypeStruct((N, 128), F32))
    in_specs.append(_full_spec(128, dnr))
    out_specs.append(_row_spec(dnr))
    out_shape.append(jax.ShapeDtypeStruct((N, dnr), F32))

    return pl.pallas_call(body, grid=(GRID,), in_specs=in_specs,
                          out_specs=out_specs, out_shape=out_shape)


_mid2 = _make_mid(True, 128)
_mid3 = _make_mid(False, 64)


def _a4_body(a0, a1, c0, c1, xr, b, wl, z_ref, y_ref):
    agg = a0[...] + a1[...]
    cnt = (c0[...] + c1[...])[:, :1]
    inv = 1.0 / jnp.maximum(cnt, 1.0)
    z = _mm(agg * inv, wl[...]) + xr[...] + b[...]
    z_ref[...] = z
    m = jnp.max(z, axis=1, keepdims=True)
    e = jnp.exp(z - m)
    lse = jnp.log(jnp.sum(e, axis=1, keepdims=True)) + m
    y_ref[...] = z - lse


_a4 = pl.pallas_call(
    _a4_body,
    grid=(GRID,),
    in_specs=[_row_spec(128), _row_spec(128), _row_spec(8), _row_spec(8),
              _row_spec(64), _full_spec(1, 64), _full_spec(128, 64)],
    out_specs=[_row_spec(64), _row_spec(64)],
    out_shape=[jax.ShapeDtypeStruct((N, 64), F32),
               jax.ShapeDtypeStruct((N, 64), F32)],
)


def kernel(x, edge_index, W1_l, b1, W1_r, W2_l, b2, W2_r, W3_l, b3, W3_r,
           bn1_gamma, bn1_beta, bn1_mean, bn1_var,
           bn2_gamma, bn2_beta, bn2_mean, bn2_var):
    src = edge_index[0]
    dst = edge_index[1]
    pad = EPAD - E
    pad_src = (jnp.arange(pad, dtype=jnp.int32) * 37) % N
    pad_dst = N + (jnp.arange(pad, dtype=jnp.int32) % (NPAD - N))
    srcp = jnp.concatenate([src, pad_src])
    dstp = jnp.concatenate([dst, pad_dst])
    srcp = srcp.reshape(NW, NHALF, HNCH, CH)
    dstp = dstp.reshape(NW, NHALF, HNCH, CH)
    zrows = jnp.zeros((STR, D), F32)
    ones128 = jnp.ones((CH, D), F32)

    xl1, xr1 = _a1(x, W1_l.T, W1_r.T)
    aa, cc = _sc_agg(True)(xl1, srcp, dstp, zrows, ones128)
    c8 = cc[:, :, :8]
    c0, c1 = c8[0], c8[1]
    a0, a1 = aa[0], aa[1]
    S1, xl2, xr2 = _mid2(a0, a1, c0, c1, xr1, b1[None],
                         bn1_gamma[None], bn1_beta[None],
                         bn1_mean[None], bn1_var[None],
                         W2_l.T, W2_r.T)
    aa = _sc_agg()(xl2, srcp, dstp, zrows)
    a0, a1 = aa[0], aa[1]
    S2, xr3 = _mid3(a0, a1, c0, c1, xr2, b2[None],
                    bn2_gamma[None], bn2_beta[None],
                    bn2_mean[None], bn2_var[None],
                    W3_r.T)
    aa = _sc_agg()(S2, srcp, dstp, zrows)
    z, y_pred = _a4(aa[0], aa[1], c0, c1, xr3, b3[None], W3_l.T)
    return (z, y_pred, S1, S2)
